# baseline (speedup 1.0000x reference)
import jax
import jax.numpy as jnp
from jax.experimental import pallas as pl
from jax.experimental.pallas import tpu as pltpu

_NCH = 16
_B = 16384
_CH = _B // _NCH


def _mlp_kernel(x_hbm, w1_ref, b1_ref, w2_ref, b2_ref, o_ref, xbuf, sem):
    for c in range(_NCH):
        pltpu.make_async_copy(
            x_hbm.at[pl.ds(c * _CH, _CH), :], xbuf.at[c], sem.at[c]
        ).start()
    w1 = w1_ref[...]
    b1v = b1_ref[...]
    w2 = w2_ref[...]
    b2v = b2_ref[...]
    for c in range(_NCH):
        pltpu.make_async_copy(
            x_hbm.at[pl.ds(c * _CH, _CH), :], xbuf.at[c], sem.at[c]
        ).wait()
        h = jnp.dot(xbuf[c], w1, preferred_element_type=jnp.float32)
        h = jnp.maximum(h + b1v, 0.0)
        logit = jnp.sum(h * w2, axis=1, keepdims=True) + b2v
        o_ref[pl.ds(c * _CH, _CH), :] = 0.5 * jnp.tanh(0.5 * logit) + 0.5


@jax.jit
def kernel(x, W1, b1, W2, b2):
    B, D = x.shape
    H = W1.shape[1]
    b1r = b1.reshape(1, H)
    w2r = W2.reshape(1, H)
    b2r = b2.reshape(1, 1)
    out = pl.pallas_call(
        _mlp_kernel,
        in_specs=[
            pl.BlockSpec(memory_space=pl.ANY),
            pl.BlockSpec((D, H), lambda: (0, 0)),
            pl.BlockSpec((1, H), lambda: (0, 0)),
            pl.BlockSpec((1, H), lambda: (0, 0)),
            pl.BlockSpec((1, 1), lambda: (0, 0)),
        ],
        out_specs=pl.BlockSpec((B, 1), lambda: (0, 0)),
        out_shape=jax.ShapeDtypeStruct((B, 1), jnp.float32),
        scratch_shapes=[
            pltpu.VMEM((_NCH, _CH, 128), jnp.float32),
            pltpu.SemaphoreType.DMA((_NCH,)),
        ],
    )(x, W1, b1r, w2r, b2r)
    return out
